# baseline (speedup 1.0000x reference)
import functools
import math

import jax
import jax.numpy as jnp
from jax import lax
from jax.experimental import pallas as pl
from jax.experimental.pallas import tpu as pltpu
from jax.experimental.pallas import tpu_sc as plsc

_L = 16
_LOG01 = math.log(0.1)
_NEG = float("-inf")


def _insert4(t0, t1, t2, t3, x):
  m0 = jnp.maximum(t0, x)
  w = jnp.minimum(t0, x)
  m1 = jnp.maximum(t1, w)
  w = jnp.minimum(t1, w)
  m2 = jnp.maximum(t2, w)
  w = jnp.minimum(t2, w)
  m3 = jnp.maximum(t3, w)
  return m0, m1, m2, m3


def _merge4(a, b):
  a0, a1, a2, a3 = a
  b0, b1, b2, b3 = b
  h0 = jnp.maximum(a0, b3)
  h1 = jnp.maximum(a1, b2)
  h2 = jnp.maximum(a2, b1)
  h3 = jnp.maximum(a3, b0)
  p0 = jnp.maximum(h0, h2)
  p2 = jnp.minimum(h0, h2)
  p1 = jnp.maximum(h1, h3)
  p3 = jnp.minimum(h1, h3)
  t0 = jnp.maximum(p0, p1)
  t1 = jnp.minimum(p0, p1)
  t2 = jnp.maximum(p2, p3)
  t3 = jnp.minimum(p2, p3)
  return t0, t1, t2, t3


def _build(vocab, batch, n_chunks, unroll, rep_unroll, interpret=False):
  chunk = vocab // n_chunks
  assert chunk * n_chunks == vocab and chunk % (_L * unroll) == 0
  vecs = chunk // _L
  inner = vecs // unroll
  rep_inner = vecs // rep_unroll
  assert rep_inner * rep_unroll == vecs
  assert n_chunks % 2 == 0 and chunk % 8 == 0

  info = plsc.get_sparse_core_info()
  num_cores, num_subcores = info.num_cores, info.num_subcores
  nw = num_cores * num_subcores
  rows_per_tile = batch // nw
  assert rows_per_tile * nw == batch

  mesh = plsc.VectorSubcoreMesh(
      core_axis_name="c", subcore_axis_name="s",
      num_cores=num_cores, num_subcores=num_subcores)

  @functools.partial(
      pl.kernel,
      out_type=jax.ShapeDtypeStruct((batch, _L), jnp.float32),
      mesh=mesh,
      scratch_types=[
          pltpu.VMEM((chunk,), jnp.float32),
          pltpu.VMEM((chunk,), jnp.float32),
          pltpu.VMEM((_L,), jnp.float32),
          pltpu.VMEM((_L,), jnp.float32),
          pltpu.SMEM((n_chunks,), jnp.float32),
          pltpu.SMEM((n_chunks,), jnp.int32),
          pltpu.SemaphoreType.DMA,
          pltpu.SemaphoreType.DMA,
      ],
      compiler_params=pltpu.CompilerParams(needs_layout_passes=False),
      interpret=interpret,
  )
  def sc_kernel(x_hbm, out_hbm, buf0, buf1, head, outb, cmax, cand, sem0,
                sem1):
    wid = lax.axis_index("s") * num_cores + lax.axis_index("c")
    neg_v = jnp.full((_L,), _NEG, dtype=jnp.float32)
    ii = lax.iota(jnp.int32, _L)
    iif = ii.astype(jnp.float32)
    bufs = (buf0, buf1)
    sems = (sem0, sem1)

    for r in range(rows_per_tile):
      row = wid * rows_per_tile + r
      base = row * vocab

      pltpu.sync_copy(x_hbm.at[pl.ds(base, _L)], head)
      fvec = head[...]
      m0 = jnp.max(fvec)
      mv = jnp.broadcast_to(m0, (_L,))

      pltpu.async_copy(x_hbm.at[pl.ds(base, chunk)], buf0, sem0)
      pltpu.async_copy(x_hbm.at[pl.ds(base + chunk, chunk)], buf1, sem1)

      def chunk_step(c, p, carry):
        mvc, svs = carry
        src = x_hbm.at[pl.ds(base + c * chunk, chunk)]
        pltpu.make_async_copy(src, bufs[p], sems[p]).wait()

        def inner_body(j, car):
          cms, sv2 = list(car[0]), list(car[1])
          off = j * (unroll * _L)
          for u in range(unroll):
            bk = u % 5
            x = bufs[p][pl.ds(off + u * _L, _L)]
            cms[bk] = jnp.maximum(cms[bk], x)
            sv2[bk] = sv2[bk] + jnp.exp(x - mvc)
          return tuple(cms), tuple(sv2)

        cms0 = (neg_v,) * 5
        cms, svs = lax.fori_loop(0, inner, inner_body, (cms0, svs),
                                 unroll=False)

        @pl.when(c + 2 < n_chunks)
        def _():
          nxt = x_hbm.at[pl.ds(base + (c + 2) * chunk, chunk)]
          pltpu.async_copy(nxt, bufs[p], sems[p])

        cm = jnp.maximum(jnp.maximum(cms[0], cms[1]),
                         jnp.maximum(jnp.maximum(cms[2], cms[3]), cms[4]))
        mc = jnp.max(cm)
        cmax[c] = mc
        mvn = jnp.maximum(mvc, jnp.broadcast_to(mc, (_L,)))
        scale = jnp.exp(mvc - mvn)
        svs = tuple(s * scale for s in svs)
        return mvn, svs

      def outer_body(i, carry):
        carry = chunk_step(2 * i, 0, carry)
        carry = chunk_step(2 * i + 1, 1, carry)
        return carry

      zeros = jnp.zeros((_L,), jnp.float32)
      mv, svs = lax.fori_loop(0, n_chunks // 2, outer_body,
                              (mv, (zeros,) * 5))
      sv = ((svs[0] + svs[1]) + (svs[2] + svs[3])) + svs[4]
      s_tot = jnp.broadcast_to(jnp.sum(sv), (_L,))

      def cmax_body(c, car):
        c1, c2, c3, c4 = car
        v = cmax[c]
        a1 = jnp.maximum(c1, v)
        w = jnp.minimum(c1, v)
        a2 = jnp.maximum(c2, w)
        w = jnp.minimum(c2, w)
        a3 = jnp.maximum(c3, w)
        w = jnp.minimum(c3, w)
        a4 = jnp.maximum(c4, w)
        return a1, a2, a3, a4

      ninf = jnp.float32(_NEG)
      _, _, _, c4 = lax.fori_loop(0, n_chunks, cmax_body,
                                  (ninf, ninf, ninf, ninf))

      def collect_body(c, k):
        hit = cmax[c] >= c4

        @pl.when(hit)
        def _():
          cand[k] = c

        return k + jnp.where(hit, 1, 0)

      nc = lax.fori_loop(0, n_chunks, collect_body, jnp.int32(0))

      def rep_body(i, car):
        ta, tb = car
        c = cand[i]
        pltpu.sync_copy(x_hbm.at[pl.ds(base + c * chunk, chunk)], buf0)

        def rep_inner_body(j, car2):
          ta2, tb2 = car2
          off = j * (rep_unroll * _L)
          for u in range(rep_unroll):
            x = buf0[pl.ds(off + u * _L, _L)]
            if u % 2 == 0:
              ta2 = _insert4(*ta2, x)
            else:
              tb2 = _insert4(*tb2, x)
          return ta2, tb2

        return lax.fori_loop(0, rep_inner, rep_inner_body, (ta, tb),
                             unroll=False)

      t4 = ((neg_v,) * 4, (neg_v,) * 4)
      ta, tb = lax.fori_loop(0, nc, rep_body, t4)
      t0, t1, t2, t3 = _merge4(ta, tb)

      gv = neg_v
      for kk in range(4):
        gk = jnp.broadcast_to(jnp.max(t0), (_L,))
        eq = t0 == gk
        first = ii == plsc.all_reduce_ffs(eq)
        t0 = jnp.where(first, t1, t0)
        t1 = jnp.where(first, t2, t1)
        t2 = jnp.where(first, t3, t2)
        t3 = jnp.where(first, neg_v, t3)
        gv = jnp.where(ii == kk, gk, gv)

      thr = mv + _LOG01
      roll_keys = jnp.where(ii < 12, ii + 4, ii - 12)
      _, rolled_f = plsc.sort_key_val(roll_keys, fvec)
      fb_key = jnp.float32(-1e37) - (iif - 4.0) * jnp.float32(4e30)
      pool_key = jnp.where(
          ii < 4,
          jnp.where(gv >= thr, gv, neg_v),
          jnp.where(rolled_f < thr, fb_key, neg_v),
      )
      pool_val = jnp.where(ii < 4, gv, rolled_f)
      _, sel_val = plsc.sort_key_val(pool_key, pool_val, descending=True)
      vkey = jnp.where(ii < 4, sel_val, neg_v)
      _, chosen = plsc.sort_key_val(vkey, vkey, descending=True)

      bits = plsc.bitcast(s_tot, jnp.int32).astype(jnp.float32)
      y = bits * jnp.float32(8.2629582e-8) - jnp.float32(88.05947)
      for _ in range(3):
        y = y - 1.0 + s_tot * jnp.exp(-y)
      lse = mv + y

      outb[...] = jnp.where(ii < 4, chosen - lse, 0.0)
      pltpu.sync_copy(outb, out_hbm.at[row])

  return sc_kernel


@jax.jit
def kernel(logits, scores, beams):
  del beams
  batch, vocab = logits.shape
  sc = _build(vocab, batch, n_chunks=100, unroll=25, rep_unroll=5)
  out = sc(logits.reshape(-1))
  return out[:, :4].reshape(-1) + jnp.repeat(scores, 4)


# speedup vs baseline: 1.0076x; 1.0076x over previous
import functools
import math

import jax
import jax.numpy as jnp
from jax import lax
from jax.experimental import pallas as pl
from jax.experimental.pallas import tpu as pltpu
from jax.experimental.pallas import tpu_sc as plsc

_L = 16
_LOG01 = math.log(0.1)
_NEG = float("-inf")


def _insert4(t0, t1, t2, t3, x):
  m0 = jnp.maximum(t0, x)
  w = jnp.minimum(t0, x)
  m1 = jnp.maximum(t1, w)
  w = jnp.minimum(t1, w)
  m2 = jnp.maximum(t2, w)
  w = jnp.minimum(t2, w)
  m3 = jnp.maximum(t3, w)
  return m0, m1, m2, m3


def _merge4(a, b):
  a0, a1, a2, a3 = a
  b0, b1, b2, b3 = b
  h0 = jnp.maximum(a0, b3)
  h1 = jnp.maximum(a1, b2)
  h2 = jnp.maximum(a2, b1)
  h3 = jnp.maximum(a3, b0)
  p0 = jnp.maximum(h0, h2)
  p2 = jnp.minimum(h0, h2)
  p1 = jnp.maximum(h1, h3)
  p3 = jnp.minimum(h1, h3)
  t0 = jnp.maximum(p0, p1)
  t1 = jnp.minimum(p0, p1)
  t2 = jnp.maximum(p2, p3)
  t3 = jnp.minimum(p2, p3)
  return t0, t1, t2, t3


def _build(vocab, batch, n_chunks, unroll, rep_unroll, nbuf=2,
           interpret=False):
  chunk = vocab // n_chunks
  assert chunk * n_chunks == vocab and chunk % (_L * unroll) == 0
  vecs = chunk // _L
  inner = vecs // unroll
  rep_inner = vecs // rep_unroll
  assert rep_inner * rep_unroll == vecs
  assert n_chunks % nbuf == 0 and chunk % 8 == 0

  info = plsc.get_sparse_core_info()
  num_cores, num_subcores = info.num_cores, info.num_subcores
  nw = num_cores * num_subcores
  rows_per_tile = batch // nw
  assert rows_per_tile * nw == batch

  mesh = plsc.VectorSubcoreMesh(
      core_axis_name="c", subcore_axis_name="s",
      num_cores=num_cores, num_subcores=num_subcores)

  @functools.partial(
      pl.kernel,
      out_type=jax.ShapeDtypeStruct((batch, _L), jnp.float32),
      mesh=mesh,
      scratch_types=(
          [pltpu.VMEM((chunk,), jnp.float32) for _ in range(nbuf)]
          + [
              pltpu.VMEM((_L,), jnp.float32),
              pltpu.VMEM((_L,), jnp.float32),
              pltpu.SMEM((n_chunks,), jnp.float32),
              pltpu.SMEM((n_chunks,), jnp.int32),
          ]
          + [pltpu.SemaphoreType.DMA for _ in range(nbuf)]
      ),
      compiler_params=pltpu.CompilerParams(needs_layout_passes=False),
      interpret=interpret,
  )
  def sc_kernel(x_hbm, out_hbm, *scratch):
    bufs = scratch[:nbuf]
    head, outb, cmax, cand = scratch[nbuf:nbuf + 4]
    sems = scratch[nbuf + 4:]
    wid = lax.axis_index("s") * num_cores + lax.axis_index("c")
    neg_v = jnp.full((_L,), _NEG, dtype=jnp.float32)
    ii = lax.iota(jnp.int32, _L)
    iif = ii.astype(jnp.float32)

    for r in range(rows_per_tile):
      row = wid * rows_per_tile + r
      base = row * vocab

      pltpu.sync_copy(x_hbm.at[pl.ds(base, _L)], head)
      fvec = head[...]
      m0 = jnp.max(fvec)
      mv = jnp.broadcast_to(m0, (_L,))

      for p in range(nbuf):
        pltpu.async_copy(x_hbm.at[pl.ds(base + p * chunk, chunk)], bufs[p],
                         sems[p])

      def chunk_step(c, p, carry):
        mvc, svs = carry
        src = x_hbm.at[pl.ds(base + c * chunk, chunk)]
        pltpu.make_async_copy(src, bufs[p], sems[p]).wait()

        def inner_body(j, car):
          cms, sv2 = list(car[0]), list(car[1])
          off = j * (unroll * _L)
          for u in range(unroll):
            bk = u % 5
            x = bufs[p][pl.ds(off + u * _L, _L)]
            cms[bk] = jnp.maximum(cms[bk], x)
            sv2[bk] = sv2[bk] + jnp.exp(x - mvc)
          return tuple(cms), tuple(sv2)

        cms0 = (neg_v,) * 5
        cms, svs = lax.fori_loop(0, inner, inner_body, (cms0, svs),
                                 unroll=False)

        @pl.when(c + nbuf < n_chunks)
        def _():
          nxt = x_hbm.at[pl.ds(base + (c + nbuf) * chunk, chunk)]
          pltpu.async_copy(nxt, bufs[p], sems[p])

        cm = jnp.maximum(jnp.maximum(cms[0], cms[1]),
                         jnp.maximum(jnp.maximum(cms[2], cms[3]), cms[4]))
        mc = jnp.max(cm)
        cmax[c] = mc
        mvn = jnp.maximum(mvc, jnp.broadcast_to(mc, (_L,)))
        scale = jnp.exp(mvc - mvn)
        svs = tuple(s * scale for s in svs)
        return mvn, svs

      def outer_body(i, carry):
        for p in range(nbuf):
          carry = chunk_step(nbuf * i + p, p, carry)
        return carry

      zeros = jnp.zeros((_L,), jnp.float32)
      mv, svs = lax.fori_loop(0, n_chunks // nbuf, outer_body,
                              (mv, (zeros,) * 5))
      sv = ((svs[0] + svs[1]) + (svs[2] + svs[3])) + svs[4]
      s_tot = jnp.broadcast_to(jnp.sum(sv), (_L,))

      def cmax_body(c, car):
        c1, c2, c3, c4 = car
        v = cmax[c]
        a1 = jnp.maximum(c1, v)
        w = jnp.minimum(c1, v)
        a2 = jnp.maximum(c2, w)
        w = jnp.minimum(c2, w)
        a3 = jnp.maximum(c3, w)
        w = jnp.minimum(c3, w)
        a4 = jnp.maximum(c4, w)
        return a1, a2, a3, a4

      ninf = jnp.float32(_NEG)
      _, _, _, c4 = lax.fori_loop(0, n_chunks, cmax_body,
                                  (ninf, ninf, ninf, ninf))

      def collect_body(c, k):
        hit = cmax[c] >= c4

        @pl.when(hit)
        def _():
          cand[k] = c

        return k + jnp.where(hit, 1, 0)

      nc = lax.fori_loop(0, n_chunks, collect_body, jnp.int32(0))

      def rep_body(i, car):
        ta, tb = car
        c = cand[i]
        pltpu.sync_copy(x_hbm.at[pl.ds(base + c * chunk, chunk)], bufs[0])

        def rep_inner_body(j, car2):
          ta2, tb2 = car2
          off = j * (rep_unroll * _L)
          for u in range(rep_unroll):
            x = bufs[0][pl.ds(off + u * _L, _L)]
            if u % 2 == 0:
              ta2 = _insert4(*ta2, x)
            else:
              tb2 = _insert4(*tb2, x)
          return ta2, tb2

        return lax.fori_loop(0, rep_inner, rep_inner_body, (ta, tb),
                             unroll=False)

      t4 = ((neg_v,) * 4, (neg_v,) * 4)
      ta, tb = lax.fori_loop(0, nc, rep_body, t4)
      t0, t1, t2, t3 = _merge4(ta, tb)

      gv = neg_v
      for kk in range(4):
        gk = jnp.broadcast_to(jnp.max(t0), (_L,))
        eq = t0 == gk
        first = ii == plsc.all_reduce_ffs(eq)
        t0 = jnp.where(first, t1, t0)
        t1 = jnp.where(first, t2, t1)
        t2 = jnp.where(first, t3, t2)
        t3 = jnp.where(first, neg_v, t3)
        gv = jnp.where(ii == kk, gk, gv)

      thr = mv + _LOG01
      roll_keys = jnp.where(ii < 12, ii + 4, ii - 12)
      _, rolled_f = plsc.sort_key_val(roll_keys, fvec)
      fb_key = jnp.float32(-1e37) - (iif - 4.0) * jnp.float32(4e30)
      pool_key = jnp.where(
          ii < 4,
          jnp.where(gv >= thr, gv, neg_v),
          jnp.where(rolled_f < thr, fb_key, neg_v),
      )
      pool_val = jnp.where(ii < 4, gv, rolled_f)
      _, sel_val = plsc.sort_key_val(pool_key, pool_val, descending=True)
      vkey = jnp.where(ii < 4, sel_val, neg_v)
      _, chosen = plsc.sort_key_val(vkey, vkey, descending=True)

      bits = plsc.bitcast(s_tot, jnp.int32).astype(jnp.float32)
      y = bits * jnp.float32(8.2629582e-8) - jnp.float32(88.05947)
      for _ in range(3):
        y = y - 1.0 + s_tot * jnp.exp(-y)
      lse = mv + y

      outb[...] = jnp.where(ii < 4, chosen - lse, 0.0)
      pltpu.sync_copy(outb, out_hbm.at[row])

  return sc_kernel


@jax.jit
def kernel(logits, scores, beams):
  del beams
  batch, vocab = logits.shape
  sc = _build(vocab, batch, n_chunks=100, unroll=25, rep_unroll=5, nbuf=10)
  out = sc(logits.reshape(-1))
  return out[:, :4].reshape(-1) + jnp.repeat(scores, 4)
